# Initial kernel scaffold; baseline (speedup 1.0000x reference)
import jax
import jax.numpy as jnp
from jax.experimental import pallas as pl


def kernel(xyz_q, lat_rep, xyz, points, sampled_grid_feat, closest_seen, fc_delta_w1, fc_delta_b1, fc_delta_w2, fc_delta_b2, fc_gamma_w1, fc_gamma_b1, fc_gamma_w2, fc_gamma_b2, w_k_global, w_v_global, w_qs, w_ks, w_vs, w_kc, w_vc):
    raise NotImplementedError("write your pallas kernel here")


# speedup vs baseline: 25.6645x; 25.6645x over previous
import functools

import jax
import jax.numpy as jnp
from jax import lax
from jax.experimental import pallas as pl
from jax.experimental.pallas import tpu as pltpu
from jax.experimental.pallas import tpu_sc as plsc

_B, _NQ, _N, _DIM = 2, 2048, 10000, 64
_K = 16
_NPAD = 10112
_QBLK_A = 128
_QC = 256
_TBL_D = 128
_ROWS = _B * _K * _NQ
_NW = 32
_CHUNK = 512
_BIG = 1e30


def _topk_body(xq_ref, xz_ref, out_ref):
    qb = xq_ref[0]
    xb = xz_ref[0]
    dot = lax.dot_general(qb, xb, (((0,), (0,)), ((), ())),
                          preferred_element_type=jnp.float32)
    q2 = jnp.sum(qb * qb, axis=0)
    x2 = jnp.sum(xb * xb, axis=0)
    d = -2.0 * dot + q2[:, None] + x2[None, :]
    iota = lax.broadcasted_iota(jnp.int32, (_QBLK_A, _NPAD), 1)
    for k in range(_K):
        m = jnp.min(d, axis=1)
        idx = jnp.min(jnp.where(d == m[:, None], iota, _NPAD), axis=1)
        out_ref[0, k, :] = idx
        d = jnp.where(iota == idx[:, None], _BIG, d)


def _topk(xq_t, xz_t):
    return pl.pallas_call(
        _topk_body,
        grid=(_B, _NQ // _QBLK_A),
        in_specs=[
            pl.BlockSpec((1, 8, _QBLK_A), lambda b, q: (b, 0, q)),
            pl.BlockSpec((1, 8, _NPAD), lambda b, q: (b, 0, 0)),
        ],
        out_specs=pl.BlockSpec((1, _K, _QBLK_A), lambda b, q: (b, 0, q)),
        out_shape=jax.ShapeDtypeStruct((_B, _K, _NQ), jnp.int32),
    )(xq_t, xz_t)


def _sc_gather(table, idx):
    info = plsc.get_sparse_core_info()
    nc = info.num_cores
    mesh = plsc.VectorSubcoreMesh(core_axis_name="c", subcore_axis_name="s")
    b_per_w = _ROWS // _NW

    @functools.partial(
        pl.kernel,
        mesh=mesh,
        out_type=jax.ShapeDtypeStruct((_ROWS, _TBL_D), jnp.float32),
        scratch_types=[
            pltpu.VMEM((_CHUNK,), jnp.int32),
            pltpu.VMEM((_CHUNK, _TBL_D), jnp.float32),
            pltpu.SemaphoreType.DMA,
        ],
    )
    def gather_k(table_hbm, idx_hbm, out_hbm, idx_v, rows_v, sem):
        wid = lax.axis_index("s") * nc + lax.axis_index("c")
        for c in range(b_per_w // _CHUNK):
            base = wid * b_per_w + c * _CHUNK
            pltpu.sync_copy(idx_hbm.at[pl.ds(base, _CHUNK)], idx_v)
            pltpu.async_copy(table_hbm.at[idx_v], rows_v, sem).wait()
            pltpu.sync_copy(rows_v, out_hbm.at[pl.ds(base, _CHUNK)])

    return gather_k(table, idx)


def _attn_body(g_ref, xq_ref, cs_ref, sgf_ref, lat_ref,
               wks_ref, wvs_ref, w1g_ref, w1p8_ref, db1_ref, dw2_ref, db2_ref,
               gw1_ref, gb1_ref, gw2_ref, gb2_ref,
               wkc_ref, wvc_ref, wqs_ref, wkg_ref, wvg_ref, out_ref):
    f32 = jnp.float32
    dot = lambda a, b: lax.dot_general(a, b, (((1,), (0,)), ((), ())),
                                       preferred_element_type=f32)
    dotT = lambda a, b: lax.dot_general(a, b, (((0,), (0,)), ((), ())),
                                        preferred_element_type=f32)
    g4 = g_ref[0]
    qb = xq_ref[0]
    cb = cs_ref[0]
    sgf = sgf_ref[0]
    lat = lat_ref[pl.ds(pl.program_id(0), 1), :]

    G = jnp.concatenate([g4[j] for j in range(_K)], axis=0)
    KN = dot(G, wks_ref[...])
    VN = dot(G, wvs_ref[...])
    GW1 = dot(G, w1g_ref[...])

    qw1 = dotT(qb, w1p8_ref[...])
    dcw1 = dotT(qb - cb, w1p8_ref[...])

    db1 = db1_ref[...]
    db2 = db2_ref[...]
    qw1t = jnp.concatenate([qw1] * _K, axis=0)
    d_in = jnp.concatenate([qw1t - GW1, dcw1], axis=0)
    P = dot(jnp.maximum(d_in + db1, 0.0), dw2_ref[...]) + db2

    k_c = dot(sgf, wkc_ref[...])
    v_c = dot(sgf, wvc_ref[...])
    qa = dot(lat, wqs_ref[...])
    kg = dot(lat, wkg_ref[...])
    vg = dot(lat, wvg_ref[...])

    nK = _K * _QC
    H = jnp.concatenate([
        qa - KN + P[:nK],
        qa - k_c + P[nK:],
        jnp.broadcast_to(qa - kg, (_QC, 64)),
    ], axis=0)
    gb1 = gb1_ref[...]
    gb2 = gb2_ref[...]
    A = dot(jnp.maximum(dot(H, gw1_ref[...]) + gb1, 0.0), gw2_ref[...]) + gb2

    m = A[:_QC]
    for j in range(1, _K + 2):
        m = jnp.maximum(m, A[j * _QC:(j + 1) * _QC])
    VP = VN + P[:nK]
    s = jnp.zeros((_QC, 64), f32)
    num = jnp.zeros((_QC, 64), f32)
    for j in range(_K):
        e = jnp.exp(A[j * _QC:(j + 1) * _QC] - m)
        s = s + e
        num = num + e * VP[j * _QC:(j + 1) * _QC]
    e_c = jnp.exp(A[nK:nK + _QC] - m)
    s = s + e_c
    num = num + e_c * (v_c + P[nK:])
    e_g = jnp.exp(A[nK + _QC:] - m)
    s = s + e_g
    num = num + e_g * vg
    out_ref[0] = num / s


def _attn(gath, xq_t, cs_t, sgf, lat_rep, weights):
    full = lambda shape: pl.BlockSpec(shape, lambda b, q: tuple(0 for _ in shape))
    w_specs = [full(w.shape) for w in weights]
    return pl.pallas_call(
        _attn_body,
        grid=(_B, _NQ // _QC),
        in_specs=[
            pl.BlockSpec((1, _K, _QC, _TBL_D), lambda b, q: (b, 0, q, 0)),
            pl.BlockSpec((1, 8, _QC), lambda b, q: (b, 0, q)),
            pl.BlockSpec((1, 8, _QC), lambda b, q: (b, 0, q)),
            pl.BlockSpec((1, _QC, _DIM), lambda b, q: (b, q, 0)),
            pl.BlockSpec((_B, _DIM), lambda b, q: (0, 0)),
        ] + w_specs,
        out_specs=pl.BlockSpec((1, _QC, _DIM), lambda b, q: (b, q, 0)),
        out_shape=jax.ShapeDtypeStruct((_B, _NQ, _DIM), jnp.float32),
    )(gath, xq_t, cs_t, sgf, lat_rep, *weights)


def kernel(xyz_q, lat_rep, xyz, points, sampled_grid_feat, closest_seen,
           fc_delta_w1, fc_delta_b1, fc_delta_w2, fc_delta_b2,
           fc_gamma_w1, fc_gamma_b1, fc_gamma_w2, fc_gamma_b2,
           w_k_global, w_v_global, w_qs, w_ks, w_vs, w_kc, w_vc):
    f32 = jnp.float32
    xq_t = jnp.zeros((_B, 8, _NQ), f32).at[:, :3, :].set(
        xyz_q.transpose(0, 2, 1))
    xz_t = jnp.full((_B, 3, _NPAD), 100.0, f32).at[:, :, :_N].set(
        xyz.transpose(0, 2, 1))
    xz_t = jnp.concatenate([xz_t, jnp.zeros((_B, 5, _NPAD), f32)], axis=1)
    knn = _topk(xq_t, xz_t)

    flat_idx = (knn + (jnp.arange(_B, dtype=jnp.int32) * _N)[:, None, None]
                ).reshape(_ROWS)
    table = jnp.concatenate(
        [points, xyz, jnp.zeros((_B, _N, _TBL_D - _DIM - 3), f32)],
        axis=2).reshape(_B * _N, _TBL_D)
    gath = _sc_gather(table, flat_idx).reshape(_B, _K, _NQ, _TBL_D)

    cs_t = jnp.zeros((_B, 8, _NQ), f32).at[:, :3, :].set(
        closest_seen.reshape(_B, _NQ, 3).transpose(0, 2, 1))
    sgf = sampled_grid_feat.reshape(_B, _NQ, _DIM)
    wks128 = jnp.zeros((_TBL_D, _DIM), f32).at[:_DIM].set(w_ks)
    wvs128 = jnp.zeros((_TBL_D, _DIM), f32).at[:_DIM].set(w_vs)
    w1g = jnp.zeros((_TBL_D, _DIM), f32).at[_DIM:_DIM + 3].set(fc_delta_w1)
    w1p8 = jnp.zeros((8, _DIM), f32).at[:3].set(fc_delta_w1)
    weights = (wks128, wvs128, w1g, w1p8,
               fc_delta_b1.reshape(1, _DIM), fc_delta_w2,
               fc_delta_b2.reshape(1, _DIM),
               fc_gamma_w1, fc_gamma_b1.reshape(1, _DIM),
               fc_gamma_w2, fc_gamma_b2.reshape(1, _DIM),
               w_kc, w_vc, w_qs, w_k_global, w_v_global)
    return _attn(gath, xq_t, cs_t, sgf, lat_rep, weights)


# speedup vs baseline: 28.3506x; 1.1047x over previous
import functools

import jax
import jax.numpy as jnp
from jax import lax
from jax.experimental import pallas as pl
from jax.experimental.pallas import tpu as pltpu
from jax.experimental.pallas import tpu_sc as plsc

_B, _NQ, _N, _DIM = 2, 2048, 10000, 64
_K = 16
_NPAD = 10112
_QBLK_A = 128
_QC = 256
_TBL_D = 128
_ROWS = _B * _K * _NQ
_NW = 32
_CHUNK = 256
_BIG = 1e30


def _topk_body(xq_ref, xz_ref, out_ref):
    qb = xq_ref[0]
    xb = xz_ref[0]
    dot = lax.dot_general(qb, xb, (((0,), (0,)), ((), ())),
                          preferred_element_type=jnp.float32)
    q2 = jnp.sum(qb * qb, axis=0)
    x2 = jnp.sum(xb * xb, axis=0)
    d = -2.0 * dot + q2[:, None] + x2[None, :]
    iota = lax.broadcasted_iota(jnp.int32, (_QBLK_A, _NPAD), 1)
    for k in range(_K):
        idx = jnp.argmin(d, axis=1).astype(jnp.int32)
        out_ref[0, k, :] = idx
        d = jnp.where(iota == idx[:, None], _BIG, d)


def _topk(xq_t, xz_t):
    return pl.pallas_call(
        _topk_body,
        grid=(_B, _NQ // _QBLK_A),
        in_specs=[
            pl.BlockSpec((1, 8, _QBLK_A), lambda b, q: (b, 0, q)),
            pl.BlockSpec((1, 8, _NPAD), lambda b, q: (b, 0, 0)),
        ],
        out_specs=pl.BlockSpec((1, _K, _QBLK_A), lambda b, q: (b, 0, q)),
        out_shape=jax.ShapeDtypeStruct((_B, _K, _NQ), jnp.int32),
    )(xq_t, xz_t)


def _sc_gather(table, idx):
    info = plsc.get_sparse_core_info()
    nc = info.num_cores
    mesh = plsc.VectorSubcoreMesh(core_axis_name="c", subcore_axis_name="s")
    b_per_w = _ROWS // _NW

    @functools.partial(
        pl.kernel,
        mesh=mesh,
        out_type=jax.ShapeDtypeStruct((_ROWS, _TBL_D), jnp.float32),
        scratch_types=[
            pltpu.VMEM((_CHUNK,), jnp.int32),
            pltpu.VMEM((_CHUNK,), jnp.int32),
            pltpu.VMEM((_CHUNK, _TBL_D), jnp.float32),
            pltpu.VMEM((_CHUNK, _TBL_D), jnp.float32),
            pltpu.SemaphoreType.DMA,
            pltpu.SemaphoreType.DMA,
        ],
    )
    def gather_k(table_hbm, idx_hbm, out_hbm, idx_v0, idx_v1,
                 rows_v0, rows_v1, gsem, osem):
        wid = lax.axis_index("s") * nc + lax.axis_index("c")
        nch = b_per_w // _CHUNK
        idx_b = (idx_v0, idx_v1)
        rows_b = (rows_v0, rows_v1)
        pltpu.sync_copy(idx_hbm.at[pl.ds(wid * b_per_w, _CHUNK)], idx_v0)
        g = pltpu.async_copy(table_hbm.at[idx_v0], rows_v0, gsem)
        for c in range(nch):
            s = c % 2
            if c + 1 < nch:
                base_n = wid * b_per_w + (c + 1) * _CHUNK
                pltpu.sync_copy(idx_hbm.at[pl.ds(base_n, _CHUNK)],
                                idx_b[1 - s])
                g_next = pltpu.async_copy(table_hbm.at[idx_b[1 - s]],
                                          rows_b[1 - s], gsem)
            g.wait()
            base = wid * b_per_w + c * _CHUNK
            o = pltpu.async_copy(rows_b[s],
                                 out_hbm.at[pl.ds(base, _CHUNK)], osem)
            if c + 1 < nch:
                g = g_next
            o.wait()

    return gather_k(table, idx)


def _attn_body(g_ref, xq_ref, cs_ref, sgf_ref, lat_ref,
               wks_ref, wvs_ref, w1g_ref, w1p8_ref, db1_ref, dw2_ref, db2_ref,
               gw1_ref, gb1_ref, gw2_ref, gb2_ref,
               wkc_ref, wvc_ref, wqs_ref, wkg_ref, wvg_ref, out_ref):
    f32 = jnp.float32
    dot = lambda a, b: lax.dot_general(a, b, (((1,), (0,)), ((), ())),
                                       preferred_element_type=f32)
    dotT = lambda a, b: lax.dot_general(a, b, (((0,), (0,)), ((), ())),
                                        preferred_element_type=f32)
    g4 = g_ref[0]
    qb = xq_ref[0]
    cb = cs_ref[0]
    sgf = sgf_ref[0]
    lat = lat_ref[pl.ds(pl.program_id(0), 1), :]

    G = jnp.concatenate([g4[j] for j in range(_K)], axis=0)
    KN = dot(G, wks_ref[...])
    VN = dot(G, wvs_ref[...])
    GW1 = dot(G, w1g_ref[...])

    qw1 = dotT(qb, w1p8_ref[...])
    dcw1 = dotT(qb - cb, w1p8_ref[...])

    db1 = db1_ref[...]
    db2 = db2_ref[...]
    qw1t = jnp.concatenate([qw1] * _K, axis=0)
    d_in = jnp.concatenate([qw1t - GW1, dcw1], axis=0)
    P = dot(jnp.maximum(d_in + db1, 0.0), dw2_ref[...]) + db2

    k_c = dot(sgf, wkc_ref[...])
    v_c = dot(sgf, wvc_ref[...])
    qa = dot(lat, wqs_ref[...])
    kg = dot(lat, wkg_ref[...])
    vg = dot(lat, wvg_ref[...])

    nK = _K * _QC
    H = jnp.concatenate([
        qa - KN + P[:nK],
        qa - k_c + P[nK:],
        jnp.broadcast_to(qa - kg, (_QC, 64)),
    ], axis=0)
    gb1 = gb1_ref[...]
    gb2 = gb2_ref[...]
    A = dot(jnp.maximum(dot(H, gw1_ref[...]) + gb1, 0.0), gw2_ref[...]) + gb2

    m = A[:_QC]
    for j in range(1, _K + 2):
        m = jnp.maximum(m, A[j * _QC:(j + 1) * _QC])
    VP = VN + P[:nK]
    s = jnp.zeros((_QC, 64), f32)
    num = jnp.zeros((_QC, 64), f32)
    for j in range(_K):
        e = jnp.exp(A[j * _QC:(j + 1) * _QC] - m)
        s = s + e
        num = num + e * VP[j * _QC:(j + 1) * _QC]
    e_c = jnp.exp(A[nK:nK + _QC] - m)
    s = s + e_c
    num = num + e_c * (v_c + P[nK:])
    e_g = jnp.exp(A[nK + _QC:] - m)
    s = s + e_g
    num = num + e_g * vg
    out_ref[0] = num / s


def _attn(gath, xq_t, cs_t, sgf, lat_rep, weights):
    full = lambda shape: pl.BlockSpec(shape, lambda b, q: tuple(0 for _ in shape))
    w_specs = [full(w.shape) for w in weights]
    return pl.pallas_call(
        _attn_body,
        grid=(_B, _NQ // _QC),
        in_specs=[
            pl.BlockSpec((1, _K, _QC, _TBL_D), lambda b, q: (b, 0, q, 0)),
            pl.BlockSpec((1, 8, _QC), lambda b, q: (b, 0, q)),
            pl.BlockSpec((1, 8, _QC), lambda b, q: (b, 0, q)),
            pl.BlockSpec((1, _QC, _DIM), lambda b, q: (b, q, 0)),
            pl.BlockSpec((_B, _DIM), lambda b, q: (0, 0)),
        ] + w_specs,
        out_specs=pl.BlockSpec((1, _QC, _DIM), lambda b, q: (b, q, 0)),
        out_shape=jax.ShapeDtypeStruct((_B, _NQ, _DIM), jnp.float32),
    )(gath, xq_t, cs_t, sgf, lat_rep, *weights)


def kernel(xyz_q, lat_rep, xyz, points, sampled_grid_feat, closest_seen,
           fc_delta_w1, fc_delta_b1, fc_delta_w2, fc_delta_b2,
           fc_gamma_w1, fc_gamma_b1, fc_gamma_w2, fc_gamma_b2,
           w_k_global, w_v_global, w_qs, w_ks, w_vs, w_kc, w_vc):
    f32 = jnp.float32
    xq_t = jnp.zeros((_B, 8, _NQ), f32).at[:, :3, :].set(
        xyz_q.transpose(0, 2, 1))
    xz_t = jnp.full((_B, 3, _NPAD), 100.0, f32).at[:, :, :_N].set(
        xyz.transpose(0, 2, 1))
    xz_t = jnp.concatenate([xz_t, jnp.zeros((_B, 5, _NPAD), f32)], axis=1)
    knn = _topk(xq_t, xz_t)

    flat_idx = (knn + (jnp.arange(_B, dtype=jnp.int32) * _N)[:, None, None]
                ).reshape(_ROWS)
    table = jnp.concatenate(
        [points, xyz, jnp.zeros((_B, _N, _TBL_D - _DIM - 3), f32)],
        axis=2).reshape(_B * _N, _TBL_D)
    gath = _sc_gather(table, flat_idx).reshape(_B, _K, _NQ, _TBL_D)

    cs_t = jnp.zeros((_B, 8, _NQ), f32).at[:, :3, :].set(
        closest_seen.reshape(_B, _NQ, 3).transpose(0, 2, 1))
    sgf = sampled_grid_feat.reshape(_B, _NQ, _DIM)
    wks128 = jnp.zeros((_TBL_D, _DIM), f32).at[:_DIM].set(w_ks)
    wvs128 = jnp.zeros((_TBL_D, _DIM), f32).at[:_DIM].set(w_vs)
    w1g = jnp.zeros((_TBL_D, _DIM), f32).at[_DIM:_DIM + 3].set(fc_delta_w1)
    w1p8 = jnp.zeros((8, _DIM), f32).at[:3].set(fc_delta_w1)
    weights = (wks128, wvs128, w1g, w1p8,
               fc_delta_b1.reshape(1, _DIM), fc_delta_w2,
               fc_delta_b2.reshape(1, _DIM),
               fc_gamma_w1, fc_gamma_b1.reshape(1, _DIM),
               fc_gamma_w2, fc_gamma_b2.reshape(1, _DIM),
               w_kc, w_vc, w_qs, w_k_global, w_v_global)
    return _attn(gath, xq_t, cs_t, sgf, lat_rep, weights)


# speedup vs baseline: 28.5869x; 1.0083x over previous
import functools

import jax
import jax.numpy as jnp
from jax import lax
from jax.experimental import pallas as pl
from jax.experimental.pallas import tpu as pltpu
from jax.experimental.pallas import tpu_sc as plsc

_B, _NQ, _N, _DIM = 2, 2048, 10000, 64
_K = 16
_NPAD = 10112
_QBLK_A = 128
_QC = 512
_TBL_D = 128
_ROWS = _B * _K * _NQ
_NW = 32
_CHUNK = 256
_BIG = 1e30


def _topk_body(xq_ref, xz_ref, out_ref):
    qb = xq_ref[0]
    xb = xz_ref[0]
    dot = lax.dot_general(qb * -2.0, xb, (((0,), (0,)), ((), ())),
                          preferred_element_type=jnp.float32)
    q2 = jnp.sum(qb * qb, axis=0)
    x2 = jnp.sum(xb * xb, axis=0)
    d = dot + q2[:, None] + x2[None, :]
    iota = lax.broadcasted_iota(jnp.int32, (_QBLK_A, _NPAD), 1)
    for k in range(_K):
        idx = jnp.argmin(d, axis=1).astype(jnp.int32)
        out_ref[0, k, :] = idx
        if k + 1 < _K:
            d = jnp.where(iota == idx[:, None], _BIG, d)


def _topk(xq_t, xz_t):
    return pl.pallas_call(
        _topk_body,
        grid=(_B, _NQ // _QBLK_A),
        in_specs=[
            pl.BlockSpec((1, 8, _QBLK_A), lambda b, q: (b, 0, q)),
            pl.BlockSpec((1, 8, _NPAD), lambda b, q: (b, 0, 0)),
        ],
        out_specs=pl.BlockSpec((1, _K, _QBLK_A), lambda b, q: (b, 0, q)),
        out_shape=jax.ShapeDtypeStruct((_B, _K, _NQ), jnp.int32),
    )(xq_t, xz_t)


def _sc_gather(table, idx):
    info = plsc.get_sparse_core_info()
    nc = info.num_cores
    mesh = plsc.VectorSubcoreMesh(core_axis_name="c", subcore_axis_name="s")
    b_per_w = _ROWS // _NW

    @functools.partial(
        pl.kernel,
        mesh=mesh,
        out_type=jax.ShapeDtypeStruct((_ROWS, _TBL_D), jnp.float32),
        scratch_types=[
            pltpu.VMEM((_CHUNK,), jnp.int32),
            pltpu.VMEM((_CHUNK,), jnp.int32),
            pltpu.VMEM((_CHUNK, _TBL_D), jnp.float32),
            pltpu.VMEM((_CHUNK, _TBL_D), jnp.float32),
            pltpu.SemaphoreType.DMA,
            pltpu.SemaphoreType.DMA,
        ],
    )
    def gather_k(table_hbm, idx_hbm, out_hbm, idx_v0, idx_v1,
                 rows_v0, rows_v1, gsem, osem):
        wid = lax.axis_index("s") * nc + lax.axis_index("c")
        nch = b_per_w // _CHUNK
        idx_b = (idx_v0, idx_v1)
        rows_b = (rows_v0, rows_v1)
        pltpu.sync_copy(idx_hbm.at[pl.ds(wid * b_per_w, _CHUNK)], idx_v0)
        g = pltpu.async_copy(table_hbm.at[idx_v0], rows_v0, gsem)
        for c in range(nch):
            s = c % 2
            if c + 1 < nch:
                base_n = wid * b_per_w + (c + 1) * _CHUNK
                pltpu.sync_copy(idx_hbm.at[pl.ds(base_n, _CHUNK)],
                                idx_b[1 - s])
                g_next = pltpu.async_copy(table_hbm.at[idx_b[1 - s]],
                                          rows_b[1 - s], gsem)
            g.wait()
            base = wid * b_per_w + c * _CHUNK
            o = pltpu.async_copy(rows_b[s],
                                 out_hbm.at[pl.ds(base, _CHUNK)], osem)
            if c + 1 < nch:
                g = g_next
            o.wait()

    return gather_k(table, idx)


def _attn_body(g_ref, xq_ref, cs_ref, sgf_ref, lat_ref,
               wks_ref, wvs_ref, w1g_ref, w1p8_ref, db1_ref, dw2_ref, db2_ref,
               gw1_ref, gb1_ref, gw2_ref, gb2_ref,
               wkc_ref, wvc_ref, wqs_ref, wkg_ref, wvg_ref, out_ref):
    f32 = jnp.float32
    dot = lambda a, b: lax.dot_general(a, b, (((1,), (0,)), ((), ())),
                                       preferred_element_type=f32)
    dotT = lambda a, b: lax.dot_general(a, b, (((0,), (0,)), ((), ())),
                                        preferred_element_type=f32)
    g4 = g_ref[0]
    qb = xq_ref[0]
    cb = cs_ref[0]
    sgf = sgf_ref[0]
    lat = lat_ref[pl.ds(pl.program_id(0), 1), :]

    G = jnp.concatenate([g4[j] for j in range(_K)], axis=0)
    KN = dot(G, wks_ref[...])
    VN = dot(G, wvs_ref[...])
    GW1 = dot(G, w1g_ref[...])

    qw1 = dotT(qb, w1p8_ref[...])
    dcw1 = dotT(qb - cb, w1p8_ref[...])

    db1 = db1_ref[...]
    db2 = db2_ref[...]
    qw1t = jnp.concatenate([qw1] * _K, axis=0)
    d_in = jnp.concatenate([qw1t - GW1, dcw1], axis=0)
    P = dot(jnp.maximum(d_in + db1, 0.0), dw2_ref[...]) + db2

    k_c = dot(sgf, wkc_ref[...])
    v_c = dot(sgf, wvc_ref[...])
    qa = dot(lat, wqs_ref[...])
    kg = dot(lat, wkg_ref[...])
    vg = dot(lat, wvg_ref[...])

    nK = _K * _QC
    H = jnp.concatenate([
        qa - KN + P[:nK],
        qa - k_c + P[nK:],
        jnp.broadcast_to(qa - kg, (_QC, 64)),
    ], axis=0)
    gb1 = gb1_ref[...]
    gb2 = gb2_ref[...]
    A = dot(jnp.maximum(dot(H, gw1_ref[...]) + gb1, 0.0), gw2_ref[...]) + gb2

    m = A[:_QC]
    for j in range(1, _K + 2):
        m = jnp.maximum(m, A[j * _QC:(j + 1) * _QC])
    VP = VN + P[:nK]
    s = jnp.zeros((_QC, 64), f32)
    num = jnp.zeros((_QC, 64), f32)
    for j in range(_K):
        e = jnp.exp(A[j * _QC:(j + 1) * _QC] - m)
        s = s + e
        num = num + e * VP[j * _QC:(j + 1) * _QC]
    e_c = jnp.exp(A[nK:nK + _QC] - m)
    s = s + e_c
    num = num + e_c * (v_c + P[nK:])
    e_g = jnp.exp(A[nK + _QC:] - m)
    s = s + e_g
    num = num + e_g * vg
    out_ref[0] = num / s


def _attn(gath, xq_t, cs_t, sgf, lat_rep, weights):
    full = lambda shape: pl.BlockSpec(shape, lambda b, q: tuple(0 for _ in shape))
    w_specs = [full(w.shape) for w in weights]
    return pl.pallas_call(
        _attn_body,
        grid=(_B, _NQ // _QC),
        in_specs=[
            pl.BlockSpec((1, _K, _QC, _TBL_D), lambda b, q: (b, 0, q, 0)),
            pl.BlockSpec((1, 8, _QC), lambda b, q: (b, 0, q)),
            pl.BlockSpec((1, 8, _QC), lambda b, q: (b, 0, q)),
            pl.BlockSpec((1, _QC, _DIM), lambda b, q: (b, q, 0)),
            pl.BlockSpec((_B, _DIM), lambda b, q: (0, 0)),
        ] + w_specs,
        out_specs=pl.BlockSpec((1, _QC, _DIM), lambda b, q: (b, q, 0)),
        out_shape=jax.ShapeDtypeStruct((_B, _NQ, _DIM), jnp.float32),
    )(gath, xq_t, cs_t, sgf, lat_rep, *weights)


def kernel(xyz_q, lat_rep, xyz, points, sampled_grid_feat, closest_seen,
           fc_delta_w1, fc_delta_b1, fc_delta_w2, fc_delta_b2,
           fc_gamma_w1, fc_gamma_b1, fc_gamma_w2, fc_gamma_b2,
           w_k_global, w_v_global, w_qs, w_ks, w_vs, w_kc, w_vc):
    f32 = jnp.float32
    xq_t = jnp.zeros((_B, 8, _NQ), f32).at[:, :3, :].set(
        xyz_q.transpose(0, 2, 1))
    xz_t = jnp.full((_B, 3, _NPAD), 100.0, f32).at[:, :, :_N].set(
        xyz.transpose(0, 2, 1))
    xz_t = jnp.concatenate([xz_t, jnp.zeros((_B, 5, _NPAD), f32)], axis=1)
    knn = _topk(xq_t, xz_t)

    flat_idx = (knn + (jnp.arange(_B, dtype=jnp.int32) * _N)[:, None, None]
                ).reshape(_ROWS)
    table = jnp.concatenate(
        [points, xyz, jnp.zeros((_B, _N, _TBL_D - _DIM - 3), f32)],
        axis=2).reshape(_B * _N, _TBL_D)
    gath = _sc_gather(table, flat_idx).reshape(_B, _K, _NQ, _TBL_D)

    cs_t = jnp.zeros((_B, 8, _NQ), f32).at[:, :3, :].set(
        closest_seen.reshape(_B, _NQ, 3).transpose(0, 2, 1))
    sgf = sampled_grid_feat.reshape(_B, _NQ, _DIM)
    wks128 = jnp.zeros((_TBL_D, _DIM), f32).at[:_DIM].set(w_ks)
    wvs128 = jnp.zeros((_TBL_D, _DIM), f32).at[:_DIM].set(w_vs)
    w1g = jnp.zeros((_TBL_D, _DIM), f32).at[_DIM:_DIM + 3].set(fc_delta_w1)
    w1p8 = jnp.zeros((8, _DIM), f32).at[:3].set(fc_delta_w1)
    weights = (wks128, wvs128, w1g, w1p8,
               fc_delta_b1.reshape(1, _DIM), fc_delta_w2,
               fc_delta_b2.reshape(1, _DIM),
               fc_gamma_w1, fc_gamma_b1.reshape(1, _DIM),
               fc_gamma_w2, fc_gamma_b2.reshape(1, _DIM),
               w_kc, w_vc, w_qs, w_k_global, w_v_global)
    return _attn(gath, xq_t, cs_t, sgf, lat_rep, weights)
